# baseline (speedup 1.0000x reference)
import jax
import jax.numpy as jnp
import numpy as np
from jax.experimental import pallas as pl
from jax.experimental.pallas import tpu as pltpu

B, T, D, DI, K, M, H, HD, L, VOCAB = 8, 512, 256, 512, 4, 2, 4, 64, 2, 272
MH = M * H
MD = M * D
PW = 128
MDP = MH * PW
HP = H * PW
F32 = jnp.float32


def _rms(v, w):
    return v * jax.lax.rsqrt(jnp.mean(v * v, axis=-1, keepdims=True) + 1e-6) * w


def _expander(rows, cols):
    r = jax.lax.broadcasted_iota(jnp.int32, (rows, cols), 0)
    c = jax.lax.broadcasted_iota(jnp.int32, (rows, cols), 1)
    return jnp.where(c // (cols // rows) == r, 1.0, 0.0).astype(F32)


def _expander_padded(rows):
    cols = rows * PW
    r = jax.lax.broadcasted_iota(jnp.int32, (rows, cols), 0)
    c = jax.lax.broadcasted_iota(jnp.int32, (rows, cols), 1)
    keep = jnp.logical_and(c // PW == r, c % PW < HD)
    return jnp.where(keep, 1.0, 0.0).astype(F32)


CHUNK = 64
NCHUNK = T // CHUNK


def _model_kernel(ids_ref, emb_ref, embT_ref, nw_ref, wup_ref, wgate_ref,
                  wdown_ref, cw_ref, cb_ref, wv_ref, wg_ref, wb_ref, wa_ref,
                  alog_ref, dtb_ref, wbl_ref, wout_ref, fnw_ref, rp_ref,
                  out_ref, VALS, RKS, WKS, CSC, BTC, BLENDC, READS,
                  A0C, A1C, PC, QC, S):
    ids = ids_ref[0]
    iota_v = jax.lax.broadcasted_iota(jnp.int32, (T, VOCAB), 1)
    onehot = jnp.where(iota_v == ids, 1.0, 0.0).astype(F32)
    x = jnp.dot(onehot, emb_ref[...], preferred_element_type=F32)

    E4 = _expander(H, D)
    E8p = _expander_padded(MH)
    E4p = _expander_padded(H)

    rr = jax.lax.broadcasted_iota(jnp.int32, (CHUNK, CHUNK), 0)
    cc = jax.lax.broadcasted_iota(jnp.int32, (CHUNK, CHUNK), 1)
    TRIL_S = jnp.where(rr > cc, 1.0, 0.0).astype(F32)
    TRIL_I = jnp.where(rr >= cc, 1.0, 0.0).astype(F32)
    I64 = jnp.where(rr == cc, 1.0, 0.0).astype(F32)
    ones_row = jnp.ones((1, CHUNK), F32)
    tr = jax.lax.broadcasted_iota(jnp.int32, (T, T), 0)
    tc = jax.lax.broadcasted_iota(jnp.int32, (T, T), 1)
    CTRI = jnp.where(jnp.logical_and(tr // CHUNK == tc // CHUNK, tr >= tc),
                     1.0, 0.0).astype(F32)

    READS[...] = jnp.zeros((T, MDP), F32)

    for l in range(L):
        nx = _rms(x, nw_ref[l])
        up = jnp.dot(nx, wup_ref[l], preferred_element_type=F32)
        y = up * cw_ref[l, K - 1]
        for j in range(1, K):
            shifted = jnp.concatenate(
                [jnp.zeros((j, DI), F32), up[:T - j]], axis=0)
            y = y + shifted * cw_ref[l, K - 1 - j]
        h = jax.nn.silu(y + cb_ref[l])
        g = jax.nn.silu(jnp.dot(nx, wgate_ref[l], preferred_element_type=F32))
        u = jnp.dot(g * h, wdown_ref[l], preferred_element_type=F32)

        VALS[...] = jnp.dot(u, wv_ref[l], preferred_element_type=F32)

        sq = u * u
        ss = jax.lax.dot_general(sq, E4, (((1,), (1,)), ((), ())),
                                 preferred_element_type=F32)
        inv = 1.0 / jnp.maximum(jnp.sqrt(ss), 1e-12)
        invb = jnp.dot(inv, E4, preferred_element_type=F32)
        rk0 = u * invb
        rks_full = jnp.dot(rk0, rp_ref[...], preferred_element_type=F32)
        RKS[...] = rks_full
        WKS[...] = jnp.concatenate(
            [jnp.zeros((1, MDP), F32), rks_full[:T - 1]], axis=0)

        bts, lds, bls = [], [], []
        for m in range(M):
            bts.append(jax.nn.sigmoid(
                jnp.dot(u, wb_ref[l, m], preferred_element_type=F32)))
            al = jnp.dot(u, wa_ref[l, m], preferred_element_type=F32) \
                + dtb_ref[l, m]
            lds.append(-jnp.exp(alog_ref[l, m]) * jax.nn.softplus(al))
            bls.append(jnp.dot(u, wbl_ref[l, m], preferred_element_type=F32))
        bt8 = jnp.concatenate(bts, axis=1)
        ld8 = jnp.concatenate(lds, axis=1)
        cs8 = jnp.dot(CTRI, ld8, preferred_element_type=F32)
        mx = jnp.maximum(bls[0], bls[1])
        e0 = jnp.exp(bls[0] - mx)
        e1 = jnp.exp(bls[1] - mx)
        sden = e0 + e1
        bl8 = jnp.concatenate([e0 / sden, e1 / sden], axis=1)
        BTC[...] = jnp.dot(bt8, E8p, preferred_element_type=F32)
        CSC[...] = jnp.dot(cs8, E8p, preferred_element_type=F32)
        BLENDC[...] = jnp.dot(bl8, E8p, preferred_element_type=F32)

        S[...] = jnp.zeros((MH, HD, HD), F32)

        def phase_a(c, carry):
            rows = pl.ds(pl.multiple_of(c * CHUNK, CHUNK), CHUNK)
            for mh in range(MH):
                lane = slice(mh * PW, mh * PW + HD)
                Kc = WKS[rows, lane]
                Vc = VALS[rows, lane]
                Rc = RKS[rows, lane]
                bt = BTC[rows, lane]
                csb = CSC[rows, lane]
                csrow = jnp.dot(ones_row, csb * I64,
                                preferred_element_type=F32)
                dcb = jnp.exp(csb)
                expo = jnp.exp(jnp.minimum(csb - csrow, 0.0))
                G = jax.lax.dot_general(Kc, Kc, (((1,), (1,)), ((), ())),
                                        preferred_element_type=F32)
                N = G * (-bt * expo * TRIL_S)
                Tm = I64 + N
                Ni = N
                for i in range(5):
                    Ni = jnp.dot(Ni, Ni, preferred_element_type=F32)
                    Tm = Tm + jnp.dot(Ni, Tm, preferred_element_type=F32)
                Kpp = Kc * (bt * dcb)
                Mc = jnp.dot(Tm, Kpp, preferred_element_type=F32)
                ev = jnp.dot(Tm, bt * Vc, preferred_element_type=F32)
                RK = jax.lax.dot_general(Rc, Kc, (((1,), (1,)), ((), ())),
                                         preferred_element_type=F32)
                Wm = RK * (expo * TRIL_I)
                A1C[rows, lane] = Rc * dcb \
                    - jnp.dot(Wm, Mc, preferred_element_type=F32)
                A0C[rows, lane] = jnp.dot(Wm, ev, preferred_element_type=F32)
                csC = csb[CHUNK - 1:CHUNK, :]
                Kd = Kc * jnp.exp(jnp.minimum(csC - csb, 0.0))
                PC[rows, lane] = jnp.exp(csC) * I64 - jax.lax.dot_general(
                    Kd, Mc, (((0,), (0,)), ((), ())),
                    preferred_element_type=F32)
                QC[rows, lane] = jax.lax.dot_general(
                    Kd, ev, (((0,), (0,)), ((), ())),
                    preferred_element_type=F32)
            return carry

        jax.lax.fori_loop(0, NCHUNK, phase_a, 0)

        def phase_b(c, carry):
            rows = pl.ds(pl.multiple_of(c * CHUNK, CHUNK), CHUNK)
            for mh in range(MH):
                lane = slice(mh * PW, mh * PW + HD)
                S0 = S[mh]
                READS[rows, lane] = jnp.dot(
                    A1C[rows, lane], S0, preferred_element_type=F32) \
                    + A0C[rows, lane]
                S[mh] = jnp.dot(PC[rows, lane], S0,
                                preferred_element_type=F32) + QC[rows, lane]
            return carry

        jax.lax.fori_loop(0, NCHUNK, phase_b, 0)

        gate = jax.nn.sigmoid(jnp.dot(u, wg_ref[l], preferred_element_type=F32))
        gb = jnp.dot(gate, E4p, preferred_element_type=F32)
        ro = READS[...] * BLENDC[...]
        o_all = (ro[:, :HP] + ro[:, HP:]) * gb
        x = x + u + jnp.dot(o_all, wout_ref[l], preferred_element_type=F32)

    fx = _rms(x, fnw_ref[...])
    out_ref[0] = jnp.dot(fx, embT_ref[...], preferred_element_type=F32)


def _full(index_dims):
    return lambda shape: pl.BlockSpec(shape, lambda b: (0,) * index_dims)


def kernel(ids, emb, norm_w, W_up, W_gate, W_down, conv_w, conv_b,
           Wv, Wg, Wb, Wa, A_log, dt_bias, Wblend, Wout, final_norm_w):
    ids3 = ids[..., None].astype(jnp.int32)
    embT = emb.T
    norm_w3 = norm_w[:, None, :]
    conv_wT = jnp.transpose(conv_w, (0, 2, 1))[:, :, None, :]
    conv_b3 = conv_b[:, None, :]
    Wblend_r = jnp.transpose(
        Wblend.reshape(L, D, H, M), (0, 3, 1, 2))
    A_log4 = A_log[:, :, None, :]
    dt_bias4 = dt_bias[:, :, None, :]
    fnw2 = final_norm_w[None, :]

    Wv_p = jnp.pad(Wv.reshape(L, D, MH, HD),
                   ((0, 0), (0, 0), (0, 0), (0, PW - HD))).reshape(L, D, MDP)
    Wout_p = jnp.pad(Wout.reshape(L, H, HD, D),
                     ((0, 0), (0, 0), (0, PW - HD), (0, 0))).reshape(L, HP, D)

    rp = np.zeros((D, MDP), np.float32)
    for h in range(H):
        for dd in range(HD):
            rp[h * HD + dd, h * PW + dd] = 1.0
        for a in range(0, HD, 2):
            rp[h * HD + a + 1, H * PW + h * PW + a] = -1.0
            rp[h * HD + a, H * PW + h * PW + a + 1] = 1.0
    rp = jnp.asarray(rp)

    in_specs = [
        pl.BlockSpec((1, T, 1), lambda b: (b, 0, 0)),
        _full(2)((VOCAB, D)),
        _full(2)((D, VOCAB)),
        _full(3)((L, 1, D)),
        _full(3)((L, D, DI)),
        _full(3)((L, D, DI)),
        _full(3)((L, DI, D)),
        _full(4)((L, K, 1, DI)),
        _full(3)((L, 1, DI)),
        _full(3)((L, D, MDP)),
        _full(3)((L, D, H)),
        _full(4)((L, M, D, H)),
        _full(4)((L, M, D, H)),
        _full(4)((L, M, 1, H)),
        _full(4)((L, M, 1, H)),
        _full(4)((L, M, D, H)),
        _full(3)((L, HP, D)),
        _full(2)((1, D)),
        _full(2)((D, MDP)),
    ]

    out = pl.pallas_call(
        _model_kernel,
        out_shape=jax.ShapeDtypeStruct((B, T, VOCAB), F32),
        grid=(B,),
        in_specs=in_specs,
        out_specs=pl.BlockSpec((1, T, VOCAB), lambda b: (b, 0, 0)),
        scratch_shapes=[
            pltpu.VMEM((T, MDP), F32),
            pltpu.VMEM((T, MDP), F32),
            pltpu.VMEM((T, MDP), F32),
            pltpu.VMEM((T, MDP), F32),
            pltpu.VMEM((T, MDP), F32),
            pltpu.VMEM((T, MDP), F32),
            pltpu.VMEM((T, MDP), F32),
            pltpu.VMEM((T, MDP), F32),
            pltpu.VMEM((T, MDP), F32),
            pltpu.VMEM((T, MDP), F32),
            pltpu.VMEM((T, MDP), F32),
            pltpu.VMEM((MH, HD, HD), F32),
        ],
        compiler_params=pltpu.CompilerParams(
            dimension_semantics=("parallel",),
            vmem_limit_bytes=56 * 1024 * 1024,
        ),
        name="fused_hebbian_mamba",
    )(ids3, emb, embT, norm_w3, W_up, W_gate, W_down, conv_wT, conv_b3,
      Wv_p, Wg, Wb, Wa, A_log4, dt_bias4, Wblend_r, Wout_p, fnw2, rp)
    return out


# speedup vs baseline: 3.7864x; 3.7864x over previous
import jax
import jax.numpy as jnp
import numpy as np
from jax.experimental import pallas as pl
from jax.experimental.pallas import tpu as pltpu

B, T, D, DI, K, M, H, HD, L, VOCAB = 8, 512, 256, 512, 4, 2, 4, 64, 2, 272
MH = M * H
MD = M * D
PW = 128
MDP = MH * PW
HP = H * PW
F32 = jnp.float32


def _rms(v, w):
    return v * jax.lax.rsqrt(jnp.mean(v * v, axis=-1, keepdims=True) + 1e-6) * w


def _expander(rows, cols):
    r = jax.lax.broadcasted_iota(jnp.int32, (rows, cols), 0)
    c = jax.lax.broadcasted_iota(jnp.int32, (rows, cols), 1)
    return jnp.where(c // (cols // rows) == r, 1.0, 0.0).astype(F32)


def _expander_padded(rows):
    cols = rows * PW
    r = jax.lax.broadcasted_iota(jnp.int32, (rows, cols), 0)
    c = jax.lax.broadcasted_iota(jnp.int32, (rows, cols), 1)
    keep = jnp.logical_and(c // PW == r, c % PW < HD)
    return jnp.where(keep, 1.0, 0.0).astype(F32)


CHUNK = 64
NCHUNK = T // CHUNK


def _model_kernel(ids_ref, emb_ref, embT_ref, nw_ref, wup_ref, wgate_ref,
                  wdown_ref, cw_ref, cb_ref, wv_ref, wg_ref, wb_ref, wa_ref,
                  alog_ref, dtb_ref, wbl_ref, wout_ref, fnw_ref, rp_ref,
                  out_ref, VALS, RKS, WKS, CSC, BTC, BLENDC, READS, S):
    ids = ids_ref[0]
    iota_v = jax.lax.broadcasted_iota(jnp.int32, (T, VOCAB), 1)
    onehot = jnp.where(iota_v == ids, 1.0, 0.0).astype(F32)
    x = jnp.dot(onehot, emb_ref[...], preferred_element_type=F32)

    E4 = _expander(H, D)
    E8p = _expander_padded(MH)
    E4p = _expander_padded(H)

    rr = jax.lax.broadcasted_iota(jnp.int32, (CHUNK, CHUNK), 0)
    cc = jax.lax.broadcasted_iota(jnp.int32, (CHUNK, CHUNK), 1)
    TRIL_S = jnp.where(rr > cc, 1.0, 0.0).astype(F32)
    TRIL_I = jnp.where(rr >= cc, 1.0, 0.0).astype(F32)
    I64 = jnp.where(rr == cc, 1.0, 0.0).astype(F32)
    ones_row = jnp.ones((1, CHUNK), F32)
    tr = jax.lax.broadcasted_iota(jnp.int32, (T, T), 0)
    tc = jax.lax.broadcasted_iota(jnp.int32, (T, T), 1)
    CTRI = jnp.where(jnp.logical_and(tr // CHUNK == tc // CHUNK, tr >= tc),
                     1.0, 0.0).astype(F32)

    READS[...] = jnp.zeros((T, MDP), F32)

    for l in range(L):
        nx = _rms(x, nw_ref[l])
        up = jnp.dot(nx, wup_ref[l], preferred_element_type=F32)
        y = up * cw_ref[l, K - 1]
        for j in range(1, K):
            shifted = jnp.concatenate(
                [jnp.zeros((j, DI), F32), up[:T - j]], axis=0)
            y = y + shifted * cw_ref[l, K - 1 - j]
        h = jax.nn.silu(y + cb_ref[l])
        g = jax.nn.silu(jnp.dot(nx, wgate_ref[l], preferred_element_type=F32))
        u = jnp.dot(g * h, wdown_ref[l], preferred_element_type=F32)

        VALS[...] = jnp.dot(u, wv_ref[l], preferred_element_type=F32)

        sq = u * u
        ss = jax.lax.dot_general(sq, E4, (((1,), (1,)), ((), ())),
                                 preferred_element_type=F32)
        inv = 1.0 / jnp.maximum(jnp.sqrt(ss), 1e-12)
        invb = jnp.dot(inv, E4, preferred_element_type=F32)
        rk0 = u * invb
        rks_full = jnp.dot(rk0, rp_ref[...], preferred_element_type=F32)
        RKS[...] = rks_full
        WKS[...] = jnp.concatenate(
            [jnp.zeros((1, MDP), F32), rks_full[:T - 1]], axis=0)

        bts, lds, bls = [], [], []
        for m in range(M):
            bts.append(jax.nn.sigmoid(
                jnp.dot(u, wb_ref[l, m], preferred_element_type=F32)))
            al = jnp.dot(u, wa_ref[l, m], preferred_element_type=F32) \
                + dtb_ref[l, m]
            lds.append(-jnp.exp(alog_ref[l, m]) * jax.nn.softplus(al))
            bls.append(jnp.dot(u, wbl_ref[l, m], preferred_element_type=F32))
        bt8 = jnp.concatenate(bts, axis=1)
        ld8 = jnp.concatenate(lds, axis=1)
        cs8 = jnp.dot(CTRI, ld8, preferred_element_type=F32)
        mx = jnp.maximum(bls[0], bls[1])
        e0 = jnp.exp(bls[0] - mx)
        e1 = jnp.exp(bls[1] - mx)
        sden = e0 + e1
        bl8 = jnp.concatenate([e0 / sden, e1 / sden], axis=1)
        BTC[...] = jnp.dot(bt8, E8p, preferred_element_type=F32)
        CSC[...] = jnp.dot(cs8, E8p, preferred_element_type=F32)
        BLENDC[...] = jnp.dot(bl8, E8p, preferred_element_type=F32)

        S[...] = jnp.zeros((MH, HD, HD), F32)

        def chunk_body(c, carry):
            rows = pl.ds(pl.multiple_of(c * CHUNK, CHUNK), CHUNK)
            lanes = [slice(mh * PW, mh * PW + HD) for mh in range(MH)]
            Kc = [WKS[rows, ln] for ln in lanes]
            Vc = [VALS[rows, ln] for ln in lanes]
            Rc = [RKS[rows, ln] for ln in lanes]
            bt = [BTC[rows, ln] for ln in lanes]
            csb = [CSC[rows, ln] for ln in lanes]
            S0 = [S[mh] for mh in range(MH)]
            csrow = [jnp.dot(ones_row, csb[m] * I64,
                             preferred_element_type=F32) for m in range(MH)]
            dcb = [jnp.exp(csb[m]) for m in range(MH)]
            expo = [jnp.exp(jnp.minimum(csb[m] - csrow[m], 0.0))
                    for m in range(MH)]
            KK = [jax.lax.dot_general(Kc[m], Kc[m], (((1,), (1,)), ((), ())),
                                      preferred_element_type=F32)
                  for m in range(MH)]
            RK = [jax.lax.dot_general(Rc[m], Kc[m], (((1,), (1,)), ((), ())),
                                      preferred_element_type=F32)
                  for m in range(MH)]
            Ni = [KK[m] * (-bt[m] * expo[m] * TRIL_S) for m in range(MH)]
            xx = [bt[m] * Vc[m]
                  - jnp.dot(Kc[m] * (bt[m] * dcb[m]), S0[m],
                            preferred_element_type=F32) for m in range(MH)]
            for i in range(6):
                xx = [xx[m] + jnp.dot(Ni[m], xx[m],
                                      preferred_element_type=F32)
                      for m in range(MH)]
                if i < 5:
                    Ni = [jnp.dot(Ni[m], Ni[m], preferred_element_type=F32)
                          for m in range(MH)]
            Wm = [RK[m] * (expo[m] * TRIL_I) for m in range(MH)]
            read = [jnp.dot(Rc[m] * dcb[m], S0[m], preferred_element_type=F32)
                    + jnp.dot(Wm[m], xx[m], preferred_element_type=F32)
                    for m in range(MH)]
            csC = [csb[m][CHUNK - 1:CHUNK, :] for m in range(MH)]
            Kd = [Kc[m] * jnp.exp(jnp.minimum(csC[m] - csb[m], 0.0))
                  for m in range(MH)]
            for mh in range(MH):
                S[mh] = jnp.exp(csC[mh]) * S0[mh] + jax.lax.dot_general(
                    Kd[mh], xx[mh], (((0,), (0,)), ((), ())),
                    preferred_element_type=F32)
                READS[rows, lanes[mh]] = read[mh]
            return carry

        jax.lax.fori_loop(0, NCHUNK, chunk_body, 0)

        gate = jax.nn.sigmoid(jnp.dot(u, wg_ref[l], preferred_element_type=F32))
        gb = jnp.dot(gate, E4p, preferred_element_type=F32)
        ro = READS[...] * BLENDC[...]
        o_all = (ro[:, :HP] + ro[:, HP:]) * gb
        x = x + u + jnp.dot(o_all, wout_ref[l], preferred_element_type=F32)

    fx = _rms(x, fnw_ref[...])
    out_ref[0] = jnp.dot(fx, embT_ref[...], preferred_element_type=F32)


def _full(index_dims):
    return lambda shape: pl.BlockSpec(shape, lambda b: (0,) * index_dims)


def kernel(ids, emb, norm_w, W_up, W_gate, W_down, conv_w, conv_b,
           Wv, Wg, Wb, Wa, A_log, dt_bias, Wblend, Wout, final_norm_w):
    ids3 = ids[..., None].astype(jnp.int32)
    embT = emb.T
    norm_w3 = norm_w[:, None, :]
    conv_wT = jnp.transpose(conv_w, (0, 2, 1))[:, :, None, :]
    conv_b3 = conv_b[:, None, :]
    Wblend_r = jnp.transpose(
        Wblend.reshape(L, D, H, M), (0, 3, 1, 2))
    A_log4 = A_log[:, :, None, :]
    dt_bias4 = dt_bias[:, :, None, :]
    fnw2 = final_norm_w[None, :]

    Wv_p = jnp.pad(Wv.reshape(L, D, MH, HD),
                   ((0, 0), (0, 0), (0, 0), (0, PW - HD))).reshape(L, D, MDP)
    Wout_p = jnp.pad(Wout.reshape(L, H, HD, D),
                     ((0, 0), (0, 0), (0, PW - HD), (0, 0))).reshape(L, HP, D)

    rp = np.zeros((D, MDP), np.float32)
    for h in range(H):
        for dd in range(HD):
            rp[h * HD + dd, h * PW + dd] = 1.0
        for a in range(0, HD, 2):
            rp[h * HD + a + 1, H * PW + h * PW + a] = -1.0
            rp[h * HD + a, H * PW + h * PW + a + 1] = 1.0
    rp = jnp.asarray(rp)

    in_specs = [
        pl.BlockSpec((1, T, 1), lambda b: (b, 0, 0)),
        _full(2)((VOCAB, D)),
        _full(2)((D, VOCAB)),
        _full(3)((L, 1, D)),
        _full(3)((L, D, DI)),
        _full(3)((L, D, DI)),
        _full(3)((L, DI, D)),
        _full(4)((L, K, 1, DI)),
        _full(3)((L, 1, DI)),
        _full(3)((L, D, MDP)),
        _full(3)((L, D, H)),
        _full(4)((L, M, D, H)),
        _full(4)((L, M, D, H)),
        _full(4)((L, M, 1, H)),
        _full(4)((L, M, 1, H)),
        _full(4)((L, M, D, H)),
        _full(3)((L, HP, D)),
        _full(2)((1, D)),
        _full(2)((D, MDP)),
    ]

    out = pl.pallas_call(
        _model_kernel,
        out_shape=jax.ShapeDtypeStruct((B, T, VOCAB), F32),
        grid=(B,),
        in_specs=in_specs,
        out_specs=pl.BlockSpec((1, T, VOCAB), lambda b: (b, 0, 0)),
        scratch_shapes=[
            pltpu.VMEM((T, MDP), F32),
            pltpu.VMEM((T, MDP), F32),
            pltpu.VMEM((T, MDP), F32),
            pltpu.VMEM((T, MDP), F32),
            pltpu.VMEM((T, MDP), F32),
            pltpu.VMEM((T, MDP), F32),
            pltpu.VMEM((T, MDP), F32),
            pltpu.VMEM((MH, HD, HD), F32),
        ],
        compiler_params=pltpu.CompilerParams(
            dimension_semantics=("parallel",),
            vmem_limit_bytes=56 * 1024 * 1024,
        ),
        name="fused_hebbian_mamba",
    )(ids3, emb, embT, norm_w3, W_up, W_gate, W_down, conv_wT, conv_b3,
      Wv_p, Wg, Wb, Wa, A_log4, dt_bias4, Wblend_r, Wout_p, fnw2, rp)
    return out
